# baseline (speedup 1.0000x reference)
import functools

import jax
import jax.numpy as jnp
from jax import lax
from jax.experimental import pallas as pl
from jax.experimental.pallas import tpu as pltpu
from jax.experimental.pallas import tpu_sc as plsc

_NC = 2
_NS = 16
_NW = _NC * _NS


def _run_sc_gather(xz, gidx):
    BT = gidx.shape[0]
    W = xz.shape[1]
    rows_w = BT // _NW
    CH = 128
    nch = rows_w // CH
    NBUF = 3

    mesh = plsc.VectorSubcoreMesh(core_axis_name="c", subcore_axis_name="s")

    @functools.partial(
        pl.kernel,
        mesh=mesh,
        out_type=jax.ShapeDtypeStruct((BT, W), jnp.float32),
        scratch_types=(
            [pltpu.VMEM((rows_w,), jnp.int32)]
            + [pltpu.VMEM((CH, W), jnp.float32)] * NBUF
            + [pltpu.SemaphoreType.DMA] * (2 * NBUF)
        ),
    )
    def sc_gather(xz_h, gidx_h, out_h, idx_v, *rest):
        bufs = rest[0:NBUF]
        gsems = rest[NBUF:2 * NBUF]
        wsems = rest[2 * NBUF:3 * NBUF]
        wid = lax.axis_index("s") * _NC + lax.axis_index("c")
        base = pl.multiple_of(wid * rows_w, rows_w)
        pltpu.sync_copy(gidx_h.at[pl.ds(base, rows_w)], idx_v)
        gcp = [None] * NBUF
        wcp = [None] * NBUF

        def start_gather(k):
            gcp[k % NBUF] = pltpu.async_copy(
                xz_h.at[idx_v.at[pl.ds(k * CH, CH)]],
                bufs[k % NBUF], gsems[k % NBUF])

        for k in range(min(NBUF - 1, nch)):
            start_gather(k)
        for j in range(nch):
            p = j % NBUF
            gcp[p].wait()
            wcp[p] = pltpu.async_copy(
                bufs[p], out_h.at[pl.ds(base + j * CH, CH)], wsems[p])
            k = j + NBUF - 1
            if k < nch:
                if wcp[k % NBUF] is not None:
                    wcp[k % NBUF].wait()
                start_gather(k)
        for j in range(max(0, nch - NBUF), nch):
            wcp[j % NBUF].wait()

    return sc_gather(xz, gidx)


def _dot(a, b):
    return jnp.dot(a, b, preferred_element_type=jnp.float32)


def _ln(h, g, be):
    mu = jnp.mean(h, axis=-1, keepdims=True)
    ms = jnp.mean(h * h, axis=-1, keepdims=True)
    var = ms - mu * mu
    return (h - mu) * lax.rsqrt(var + 1e-5) * g + be


def _pred_core(xe, c, nch, CN, wp):
    (w1c, g1, be1, w2c, g2, be2, wl, bl) = wp
    F = w2c.shape[1]
    M = CN + 2
    xb = xe.astype(jnp.bfloat16)
    one8 = jnp.ones((M, 8), jnp.bfloat16)
    xcat = jnp.concatenate([xb[0:M, :], xb[1:M + 1, :], xb[2:M + 2, :],
                            one8], 1)
    h1 = _ln(jax.nn.relu(_dot(xcat, w1c)), g1, be1)
    if c == 0:
        h1 = jnp.concatenate([jnp.zeros((1, F), jnp.float32), h1[1:]], 0)
    if c == nch - 1:
        h1 = jnp.concatenate([h1[:-1], jnp.zeros((1, F), jnp.float32)], 0)
    h1b = h1.astype(jnp.bfloat16)
    hcat = jnp.concatenate([h1b[0:CN, :], h1b[1:CN + 1, :],
                            h1b[2:CN + 2, :], one8[0:CN, :]], 1)
    h2 = _ln(jax.nn.relu(_dot(hcat, w2c)), g2, be2)
    return jax.nn.relu(_dot(h2, wl) + bl)


def _build_ext(c, nch, CN, D, make_rows):
    ztop = 2 if c == 0 else 0
    zbot = 2 if c == nch - 1 else 0
    lo = c * CN - 2 + ztop
    n = CN + 4 - ztop - zbot
    body = make_rows(lo, n)
    parts = []
    if ztop:
        parts.append(jnp.zeros((ztop, D), body.dtype))
    parts.append(body)
    if zbot:
        parts.append(jnp.zeros((zbot, D), body.dtype))
    return jnp.concatenate(parts, 0) if len(parts) > 1 else parts[0]


def _emb_rows(col_ref, blo_row, bhi_row, tab, lo, n):
    v = col_ref[0, pl.ds(lo, n), :]
    oh = jnp.logical_and(blo_row < v, v <= bhi_row)
    return _dot(oh.astype(tab.dtype), tab)


def _unpack_params(refs):
    (w1_ref, g1_ref, be1_ref, w2_ref, g2_ref, be2_ref,
     wl_ref, bl_ref) = refs
    return (w1_ref[...], g1_ref[...], be1_ref[...],
            w2_ref[...], g2_ref[...], be2_ref[...],
            wl_ref[...], bl_ref[0, 0])


def _param_specs_args(p, D, F):
    def wcat(w, b):
        kd = jnp.transpose(w, (2, 1, 0)).reshape(3 * w.shape[1], F)
        brow = jnp.concatenate(
            [b.reshape(1, F), jnp.zeros((7, F), jnp.float32)], 0)
        return jnp.concatenate([kd, brow], 0).astype(jnp.bfloat16)

    w1c = wcat(p["W1"], p["b1"])
    w2c = wcat(p["W2"], p["b2"])
    row = lambda v: v.reshape(1, -1)
    specs = [
        pl.BlockSpec((3 * D + 8, F), lambda b: (0, 0)),
        pl.BlockSpec((1, F), lambda b: (0, 0)),
        pl.BlockSpec((1, F), lambda b: (0, 0)),
        pl.BlockSpec((3 * F + 8, F), lambda b: (0, 0)),
        pl.BlockSpec((1, F), lambda b: (0, 0)),
        pl.BlockSpec((1, F), lambda b: (0, 0)),
        pl.BlockSpec((F, 1), lambda b: (0, 0)),
        pl.BlockSpec((1, 1), lambda b: (0, 0)),
    ]
    args = [w1c, row(p["g1"]), row(p["be1"]),
            w2c, row(p["g2"]), row(p["be2"]),
            p["Wl"], p["bl"].reshape(1, 1)]
    return specs, args


def _dur_prep_body(L, T, TC, zero_row, zero_pad, x_ref, d_ref, *rest):
    wp = _unpack_params(rest[:8])
    pred_ref, gidx_ref = rest[8], rest[9]
    D = x_ref.shape[2]
    b = pl.program_id(0)

    d_col = d_ref[0].astype(jnp.float32)
    row_i = lax.broadcasted_iota(jnp.int32, (L, L), 0)
    col_i = lax.broadcasted_iota(jnp.int32, (L, L), 1)
    tri = (col_i <= row_i).astype(jnp.float32)
    cum = jnp.dot(tri, d_col, preferred_element_type=jnp.float32)
    total = cum[L - 1, 0]
    ones_row = jnp.ones((1, L), jnp.bfloat16)
    for c in range(T // TC):
        t_row = (lax.broadcasted_iota(jnp.int32, (1, TC), 1)
                 + c * TC).astype(jnp.float32)
        mask = (cum <= t_row).astype(jnp.bfloat16)
        cnt = _dot(ones_row, mask).astype(jnp.int32)
        idxp = jnp.minimum(cnt, L - 1)
        valid = t_row < total
        zspread = zero_row + jnp.bitwise_and(
            lax.broadcasted_iota(jnp.int32, (1, TC), 1), zero_pad - 1)
        gidx = jnp.where(valid, b * L + idxp, zspread)
        gidx_ref[0, 0, pl.ds(c * TC, TC)] = gidx[0]

    nch = L // min(L, 1024)
    CN = L // nch
    for c in range(nch):
        xe = _build_ext(c, nch, CN, D,
                        lambda lo, n: x_ref[0, pl.ds(lo, n), :])
        pred_ref[0, pl.ds(c * CN, CN), :] = _pred_core(xe, c, nch, CN, wp)


def _run_dur_prep(x, d_target, T, zero_row, zero_pad, p):
    B, N, D = x.shape
    F = p["W1"].shape[0]
    TC = 1024
    d3 = d_target.astype(jnp.int32).reshape(B, N, 1)
    pspecs, pargs = _param_specs_args(p, D, F)
    pred, gidx = pl.pallas_call(
        functools.partial(_dur_prep_body, N, T, TC, zero_row, zero_pad),
        grid=(B,),
        in_specs=[pl.BlockSpec((1, N, D), lambda b: (b, 0, 0)),
                  pl.BlockSpec((1, N, 1), lambda b: (b, 0, 0))] + pspecs,
        out_specs=[pl.BlockSpec((1, N, 1), lambda b: (b, 0, 0)),
                   pl.BlockSpec((1, 1, T), lambda b: (b, 0, 0))],
        out_shape=[jax.ShapeDtypeStruct((B, N, 1), jnp.float32),
                   jax.ShapeDtypeStruct((B, 1, T), jnp.int32)],
    )(x, d3, *pargs)
    return pred.reshape(B, N), gidx.reshape(B * T)


def _mega_body(N, CN, NB, x_ref, ecol_ref, pcol_ref, eblo_ref, ebhi_ref,
               etab_ref, pblo_ref, pbhi_ref, ptab_ref, *rest):
    ewp = _unpack_params(rest[0:8])
    pwp = _unpack_params(rest[8:16])
    epred_ref, ppred_ref, h_ref = rest[16:19]
    D = x_ref.shape[2]
    eblo = eblo_ref[...]
    ebhi = ebhi_ref[...]
    etab = etab_ref[...].astype(jnp.bfloat16)
    pblo = pblo_ref[...]
    pbhi = pbhi_ref[...]
    ptab = ptab_ref[...].astype(jnp.bfloat16)
    nch = N // CN

    def rows_x(lo, n):
        return x_ref[0, pl.ds(lo, n), :]

    def rows_s1(lo, n):
        return rows_x(lo, n) + _emb_rows(ecol_ref, eblo, ebhi, etab, lo, n)

    for c in range(nch):
        s = c * CN
        xe = _build_ext(c, nch, CN, D, rows_x)
        epred_ref[0, pl.ds(s, CN), :] = _pred_core(xe, c, nch, CN, ewp)
        s1e = _build_ext(c, nch, CN, D, rows_s1)
        ppred_ref[0, pl.ds(s, CN), :] = _pred_core(s1e, c, nch, CN, pwp)
        h_ref[0, pl.ds(s, CN), :] = (
            s1e[2:CN + 2, :] + _emb_rows(pcol_ref, pblo, pbhi, ptab, s, CN))


def _run_mega(exp_x, e_target, p_target, eb2, pb2, etab, ptab, ep, pp):
    B, N, D = exp_x.shape
    F = ep["W1"].shape[0]
    NB = etab.shape[0]
    CN = 4096
    eblo, ebhi = eb2
    pblo, pbhi = pb2
    especs, eargs = _param_specs_args(ep, D, F)
    pspecs, pargs = _param_specs_args(pp, D, F)
    in_specs = [
        pl.BlockSpec((1, N, D), lambda b: (b, 0, 0)),
        pl.BlockSpec((1, N, 1), lambda b: (b, 0, 0)),
        pl.BlockSpec((1, N, 1), lambda b: (b, 0, 0)),
        pl.BlockSpec((1, NB), lambda b: (0, 0)),
        pl.BlockSpec((1, NB), lambda b: (0, 0)),
        pl.BlockSpec((NB, D), lambda b: (0, 0)),
        pl.BlockSpec((1, NB), lambda b: (0, 0)),
        pl.BlockSpec((1, NB), lambda b: (0, 0)),
        pl.BlockSpec((NB, D), lambda b: (0, 0)),
    ] + especs + pspecs
    args = ([exp_x, e_target.reshape(B, N, 1), p_target.reshape(B, N, 1),
             eblo.reshape(1, NB), ebhi.reshape(1, NB), etab,
             pblo.reshape(1, NB), pbhi.reshape(1, NB), ptab]
            + eargs + pargs)
    epred, ppred, h = pl.pallas_call(
        functools.partial(_mega_body, N, CN, NB),
        grid=(B,),
        in_specs=in_specs,
        out_specs=[
            pl.BlockSpec((1, N, 1), lambda b: (b, 0, 0)),
            pl.BlockSpec((1, N, 1), lambda b: (b, 0, 0)),
            pl.BlockSpec((1, N, D), lambda b: (b, 0, 0)),
        ],
        out_shape=[
            jax.ShapeDtypeStruct((B, N, 1), jnp.float32),
            jax.ShapeDtypeStruct((B, N, 1), jnp.float32),
            jax.ShapeDtypeStruct((B, N, D), jnp.float32),
        ],
    )(*args)
    return epred.reshape(B, N), ppred.reshape(B, N), h


def kernel(x, e_target, p_target, d_target, mel_max_length, params,
           energy_bins, pitch_bins):
    B, L, D = x.shape
    T = e_target.shape[1]

    def edges(bins):
        lo = jnp.concatenate([jnp.full((1,), -jnp.inf, bins.dtype), bins])
        hi = jnp.concatenate([bins, jnp.full((1,), jnp.inf, bins.dtype)])
        return lo, hi

    eb2 = edges(energy_bins)
    pb2 = edges(pitch_bins)

    zero_row = B * L
    zero_pad = 512
    xz = jnp.concatenate([x.reshape(B * L, D),
                          jnp.zeros((zero_pad, D), x.dtype)], axis=0)

    log_dur, gidx = _run_dur_prep(x, d_target, T, zero_row, zero_pad,
                                  params["dur"])
    exp_x = _run_sc_gather(xz, gidx).reshape(B, T, D)

    energy_pred, pitch_pred, h = _run_mega(
        exp_x, e_target, p_target, eb2, pb2,
        params["energy_emb"], params["pitch_emb"],
        params["energy"], params["pitch"])

    return (h, log_dur, pitch_pred, energy_pred)


# speedup vs baseline: 1.2052x; 1.2052x over previous
import functools

import jax
import jax.numpy as jnp
from jax import lax
from jax.experimental import pallas as pl
from jax.experimental.pallas import tpu as pltpu
from jax.experimental.pallas import tpu_sc as plsc

_NC = 2
_NS = 16
_NW = _NC * _NS


def _run_sc_gather(xz, gidx):
    BT = gidx.shape[0]
    W = xz.shape[1]
    rows_w = BT // _NW
    CH = 128
    nch = rows_w // CH
    NBUF = 3

    mesh = plsc.VectorSubcoreMesh(core_axis_name="c", subcore_axis_name="s")

    @functools.partial(
        pl.kernel,
        mesh=mesh,
        out_type=jax.ShapeDtypeStruct((BT, W), jnp.float32),
        scratch_types=(
            [pltpu.VMEM((rows_w,), jnp.int32)]
            + [pltpu.VMEM((CH, W), jnp.float32)] * NBUF
            + [pltpu.SemaphoreType.DMA] * (2 * NBUF)
        ),
    )
    def sc_gather(xz_h, gidx_h, out_h, idx_v, *rest):
        bufs = rest[0:NBUF]
        gsems = rest[NBUF:2 * NBUF]
        wsems = rest[2 * NBUF:3 * NBUF]
        wid = lax.axis_index("s") * _NC + lax.axis_index("c")
        base = pl.multiple_of(wid * rows_w, rows_w)
        pltpu.sync_copy(gidx_h.at[pl.ds(base, rows_w)], idx_v)
        gcp = [None] * NBUF
        wcp = [None] * NBUF

        def start_gather(k):
            gcp[k % NBUF] = pltpu.async_copy(
                xz_h.at[idx_v.at[pl.ds(k * CH, CH)]],
                bufs[k % NBUF], gsems[k % NBUF])

        for k in range(min(NBUF - 1, nch)):
            start_gather(k)
        for j in range(nch):
            p = j % NBUF
            gcp[p].wait()
            wcp[p] = pltpu.async_copy(
                bufs[p], out_h.at[pl.ds(base + j * CH, CH)], wsems[p])
            k = j + NBUF - 1
            if k < nch:
                if wcp[k % NBUF] is not None:
                    wcp[k % NBUF].wait()
                start_gather(k)
        for j in range(max(0, nch - NBUF), nch):
            wcp[j % NBUF].wait()

    return sc_gather(xz, gidx)


def _dot(a, b):
    return jnp.dot(a, b, preferred_element_type=jnp.float32)


def _ln(h, g, be):
    mu = jnp.mean(h, axis=-1, keepdims=True)
    ms = jnp.mean(h * h, axis=-1, keepdims=True)
    var = ms - mu * mu
    return (h - mu) * lax.rsqrt(var + 1e-5) * g + be


def _pred_core(xe, c, nch, CN, wp):
    (w1c, b1, g1, be1, w2c, b2, g2, be2, wl, bl) = wp
    F = w2c.shape[1]
    M = CN + 2
    xb = xe.astype(jnp.bfloat16)
    xcat = jnp.concatenate([xb[0:M, :], xb[1:M + 1, :], xb[2:M + 2, :]], 1)
    h1 = _dot(xcat, w1c) + b1
    h1 = _ln(jax.nn.relu(h1), g1, be1)
    if c == 0:
        h1 = jnp.concatenate([jnp.zeros((1, F), jnp.float32), h1[1:]], 0)
    if c == nch - 1:
        h1 = jnp.concatenate([h1[:-1], jnp.zeros((1, F), jnp.float32)], 0)
    h1b = h1.astype(jnp.bfloat16)
    hcat = jnp.concatenate([h1b[0:CN, :], h1b[1:CN + 1, :],
                            h1b[2:CN + 2, :]], 1)
    h2 = _dot(hcat, w2c) + b2
    h2 = _ln(jax.nn.relu(h2), g2, be2)
    return jax.nn.relu(_dot(h2, wl) + bl)


def _build_ext(c, nch, CN, D, make_rows):
    ztop = 2 if c == 0 else 0
    zbot = 2 if c == nch - 1 else 0
    lo = c * CN - 2 + ztop
    n = CN + 4 - ztop - zbot
    body = make_rows(lo, n)
    parts = []
    if ztop:
        parts.append(jnp.zeros((ztop, D), body.dtype))
    parts.append(body)
    if zbot:
        parts.append(jnp.zeros((zbot, D), body.dtype))
    return jnp.concatenate(parts, 0) if len(parts) > 1 else parts[0]


def _emb_rows(col_ref, blo_row, bhi_row, tab, lo, n):
    v = col_ref[0, pl.ds(lo, n), :]
    oh = jnp.logical_and(blo_row < v, v <= bhi_row)
    return _dot(oh.astype(tab.dtype), tab)


def _unpack_params(refs):
    (w1_ref, b1_ref, g1_ref, be1_ref, w2_ref, b2_ref, g2_ref, be2_ref,
     wl_ref, bl_ref) = refs
    return (w1_ref[...], b1_ref[...], g1_ref[...],
            be1_ref[...], w2_ref[...], b2_ref[...],
            g2_ref[...], be2_ref[...], wl_ref[...], bl_ref[0, 0])


def _param_specs_args(p, D, F):
    w1t = jnp.transpose(p["W1"], (2, 1, 0)).astype(
        jnp.bfloat16).reshape(3 * D, F)
    w2t = jnp.transpose(p["W2"], (2, 1, 0)).astype(
        jnp.bfloat16).reshape(3 * F, F)
    row = lambda v: v.reshape(1, -1)
    specs = [
        pl.BlockSpec((3 * D, F), lambda b: (0, 0)),
        pl.BlockSpec((1, F), lambda b: (0, 0)),
        pl.BlockSpec((1, F), lambda b: (0, 0)),
        pl.BlockSpec((1, F), lambda b: (0, 0)),
        pl.BlockSpec((3 * F, F), lambda b: (0, 0)),
        pl.BlockSpec((1, F), lambda b: (0, 0)),
        pl.BlockSpec((1, F), lambda b: (0, 0)),
        pl.BlockSpec((1, F), lambda b: (0, 0)),
        pl.BlockSpec((F, 1), lambda b: (0, 0)),
        pl.BlockSpec((1, 1), lambda b: (0, 0)),
    ]
    args = [w1t, row(p["b1"]), row(p["g1"]), row(p["be1"]),
            w2t, row(p["b2"]), row(p["g2"]), row(p["be2"]),
            p["Wl"], p["bl"].reshape(1, 1)]
    return specs, args


def _dur_prep_body(L, T, TC, zero_row, zero_pad, x_ref, d_ref, *rest):
    wp = _unpack_params(rest[:2 + 8])
    pred_ref, gidx_ref = rest[10], rest[11]
    D = x_ref.shape[2]
    b = pl.program_id(0)

    d_col = d_ref[0].astype(jnp.float32)
    row_i = lax.broadcasted_iota(jnp.int32, (L, L), 0)
    col_i = lax.broadcasted_iota(jnp.int32, (L, L), 1)
    tri = (col_i <= row_i).astype(jnp.float32)
    cum = jnp.dot(tri, d_col, preferred_element_type=jnp.float32)
    total = cum[L - 1, 0]
    ones_row = jnp.ones((1, L), jnp.bfloat16)
    for c in range(T // TC):
        t_row = (lax.broadcasted_iota(jnp.int32, (1, TC), 1)
                 + c * TC).astype(jnp.float32)
        mask = (cum <= t_row).astype(jnp.bfloat16)
        cnt = _dot(ones_row, mask).astype(jnp.int32)
        idxp = jnp.minimum(cnt, L - 1)
        valid = t_row < total
        zspread = zero_row + jnp.bitwise_and(
            lax.broadcasted_iota(jnp.int32, (1, TC), 1), zero_pad - 1)
        gidx = jnp.where(valid, b * L + idxp, zspread)
        gidx_ref[0, 0, pl.ds(c * TC, TC)] = gidx[0]

    nch = L // min(L, 1024)
    CN = L // nch
    for c in range(nch):
        xe = _build_ext(c, nch, CN, D,
                        lambda lo, n: x_ref[0, pl.ds(lo, n), :])
        pred_ref[0, pl.ds(c * CN, CN), :] = _pred_core(xe, c, nch, CN, wp)


def _run_dur_prep(x, d_target, T, zero_row, zero_pad, p):
    B, N, D = x.shape
    F = p["W1"].shape[0]
    TC = 1024
    d3 = d_target.astype(jnp.int32).reshape(B, N, 1)
    pspecs, pargs = _param_specs_args(p, D, F)
    pred, gidx = pl.pallas_call(
        functools.partial(_dur_prep_body, N, T, TC, zero_row, zero_pad),
        grid=(B,),
        in_specs=[pl.BlockSpec((1, N, D), lambda b: (b, 0, 0)),
                  pl.BlockSpec((1, N, 1), lambda b: (b, 0, 0))] + pspecs,
        out_specs=[pl.BlockSpec((1, N, 1), lambda b: (b, 0, 0)),
                   pl.BlockSpec((1, 1, T), lambda b: (b, 0, 0))],
        out_shape=[jax.ShapeDtypeStruct((B, N, 1), jnp.float32),
                   jax.ShapeDtypeStruct((B, 1, T), jnp.int32)],
    )(x, d3, *pargs)
    return pred.reshape(B, N), gidx.reshape(B * T)


def _mega_body(N, CN, NB, x_ref, ecol_ref, pcol_ref, eblo_ref, ebhi_ref,
               etab_ref, pblo_ref, pbhi_ref, ptab_ref, *rest):
    ewp = _unpack_params(rest[0:10])
    pwp = _unpack_params(rest[10:20])
    epred_ref, ppred_ref, h_ref = rest[20:23]
    D = x_ref.shape[2]
    eblo = eblo_ref[...]
    ebhi = ebhi_ref[...]
    etab = etab_ref[...].astype(jnp.bfloat16)
    pblo = pblo_ref[...]
    pbhi = pbhi_ref[...]
    ptab = ptab_ref[...].astype(jnp.bfloat16)
    nch = N // CN

    def rows_x(lo, n):
        return x_ref[0, pl.ds(lo, n), :]

    def rows_s1(lo, n):
        return rows_x(lo, n) + _emb_rows(ecol_ref, eblo, ebhi, etab, lo, n)

    for c in range(nch):
        s = c * CN
        xe = _build_ext(c, nch, CN, D, rows_x)
        epred_ref[0, pl.ds(s, CN), :] = _pred_core(xe, c, nch, CN, ewp)
        s1e = _build_ext(c, nch, CN, D, rows_s1)
        ppred_ref[0, pl.ds(s, CN), :] = _pred_core(s1e, c, nch, CN, pwp)
        h_ref[0, pl.ds(s, CN), :] = (
            s1e[2:CN + 2, :] + _emb_rows(pcol_ref, pblo, pbhi, ptab, s, CN))


def _run_mega(exp_x, e_target, p_target, eb2, pb2, etab, ptab, ep, pp):
    B, N, D = exp_x.shape
    F = ep["W1"].shape[0]
    NB = etab.shape[0]
    CN = 4096
    eblo, ebhi = eb2
    pblo, pbhi = pb2
    especs, eargs = _param_specs_args(ep, D, F)
    pspecs, pargs = _param_specs_args(pp, D, F)
    in_specs = [
        pl.BlockSpec((1, N, D), lambda b: (b, 0, 0)),
        pl.BlockSpec((1, N, 1), lambda b: (b, 0, 0)),
        pl.BlockSpec((1, N, 1), lambda b: (b, 0, 0)),
        pl.BlockSpec((1, NB), lambda b: (0, 0)),
        pl.BlockSpec((1, NB), lambda b: (0, 0)),
        pl.BlockSpec((NB, D), lambda b: (0, 0)),
        pl.BlockSpec((1, NB), lambda b: (0, 0)),
        pl.BlockSpec((1, NB), lambda b: (0, 0)),
        pl.BlockSpec((NB, D), lambda b: (0, 0)),
    ] + especs + pspecs
    args = ([exp_x, e_target.reshape(B, N, 1), p_target.reshape(B, N, 1),
             eblo.reshape(1, NB), ebhi.reshape(1, NB), etab,
             pblo.reshape(1, NB), pbhi.reshape(1, NB), ptab]
            + eargs + pargs)
    epred, ppred, h = pl.pallas_call(
        functools.partial(_mega_body, N, CN, NB),
        grid=(B,),
        in_specs=in_specs,
        out_specs=[
            pl.BlockSpec((1, N, 1), lambda b: (b, 0, 0)),
            pl.BlockSpec((1, N, 1), lambda b: (b, 0, 0)),
            pl.BlockSpec((1, N, D), lambda b: (b, 0, 0)),
        ],
        out_shape=[
            jax.ShapeDtypeStruct((B, N, 1), jnp.float32),
            jax.ShapeDtypeStruct((B, N, 1), jnp.float32),
            jax.ShapeDtypeStruct((B, N, D), jnp.float32),
        ],
    )(*args)
    return epred.reshape(B, N), ppred.reshape(B, N), h


def kernel(x, e_target, p_target, d_target, mel_max_length, params,
           energy_bins, pitch_bins):
    B, L, D = x.shape
    T = e_target.shape[1]

    def edges(bins):
        lo = jnp.concatenate([jnp.full((1,), -jnp.inf, bins.dtype), bins])
        hi = jnp.concatenate([bins, jnp.full((1,), jnp.inf, bins.dtype)])
        return lo, hi

    eb2 = edges(energy_bins)
    pb2 = edges(pitch_bins)

    zero_row = B * L
    zero_pad = 512
    xz = jnp.concatenate([x.reshape(B * L, D),
                          jnp.zeros((zero_pad, D), x.dtype)], axis=0)

    log_dur, gidx = _run_dur_prep(x, d_target, T, zero_row, zero_pad,
                                  params["dur"])
    exp_x = _run_sc_gather(xz, gidx).reshape(B, T, D)

    energy_pred, pitch_pred, h = _run_mega(
        exp_x, e_target, p_target, eb2, pb2,
        params["energy_emb"], params["pitch_emb"],
        params["energy"], params["pitch"])

    return (h, log_dur, pitch_pred, energy_pred)
